# Initial kernel scaffold; baseline (speedup 1.0000x reference)
import jax
import jax.numpy as jnp
from jax.experimental import pallas as pl


def kernel(token_indices, embeddings):
    raise NotImplementedError("write your pallas kernel here")


# speedup vs baseline: 3.3572x; 3.3572x over previous
import functools

import jax
import jax.numpy as jnp
from jax import lax
from jax.experimental import pallas as pl
from jax.experimental.pallas import tpu as pltpu
from jax.experimental.pallas import tpu_sc as plsc

VOCAB = 1000
EMBED_DIM = 64
BATCH = 1024
HIST = 200

NUM_CORES = 2
NUM_SUBCORES = 16
NW = NUM_CORES * NUM_SUBCORES
B_TOTAL = BATCH * HIST
ROWS_PER_W = B_TOTAL // NW
CHUNK = 128
NCHUNK = ROWS_PER_W // CHUNK
NBUF = 2

_mesh = plsc.VectorSubcoreMesh(core_axis_name="c", subcore_axis_name="s")


@functools.partial(
    pl.kernel,
    out_type=jax.ShapeDtypeStruct((B_TOTAL, EMBED_DIM), jnp.float32),
    mesh=_mesh,
    scratch_types=[
        pltpu.VMEM((NCHUNK, CHUNK), jnp.int32),
        pltpu.VMEM((NBUF, CHUNK, EMBED_DIM), jnp.float32),
        pltpu.SemaphoreType.DMA,
    ],
    compiler_params=pltpu.CompilerParams(use_tc_tiling_on_sc=False),
)
def _lookup(idx_hbm, table_hbm, out_hbm, idx_v, rows_v, gsem):
    wid = lax.axis_index("s") * NUM_CORES + lax.axis_index("c")
    base = wid * ROWS_PER_W
    pltpu.sync_copy(idx_hbm.at[wid], idx_v)
    my_idx = idx_v

    pltpu.async_copy(table_hbm.at[my_idx.at[0]], rows_v.at[0], gsem)

    def outer(i, carry):
        for b in range(NBUF):
            j = i * NBUF + b
            nxt = j + 1

            @pl.when(nxt < NCHUNK)
            def _():
                pltpu.async_copy(
                    table_hbm.at[my_idx.at[nxt]],
                    rows_v.at[(b + 1) % NBUF],
                    gsem,
                )

            pltpu.make_async_copy(
                table_hbm.at[my_idx.at[j]], rows_v.at[b], gsem
            ).wait()
            pltpu.sync_copy(
                rows_v.at[b], out_hbm.at[pl.ds(base + j * CHUNK, CHUNK)]
            )
        return carry

    lax.fori_loop(0, NCHUNK // NBUF, outer, 0)


def kernel(token_indices, embeddings):
    idx = token_indices.reshape(NW, NCHUNK, CHUNK)
    out = _lookup(idx, embeddings)
    return out.reshape(BATCH, HIST, EMBED_DIM)


# speedup vs baseline: 3.3951x; 1.0113x over previous
import functools

import jax
import jax.numpy as jnp
from jax import lax
from jax.experimental import pallas as pl
from jax.experimental.pallas import tpu as pltpu
from jax.experimental.pallas import tpu_sc as plsc

VOCAB = 1000
EMBED_DIM = 64
BATCH = 1024
HIST = 200

NUM_CORES = 2
NUM_SUBCORES = 16
NW = NUM_CORES * NUM_SUBCORES
B_TOTAL = BATCH * HIST
ROWS_PER_W = B_TOTAL // NW
CHUNK = 640
NCHUNK = ROWS_PER_W // CHUNK
NBUF = 2

_mesh = plsc.VectorSubcoreMesh(core_axis_name="c", subcore_axis_name="s")


@functools.partial(
    pl.kernel,
    out_type=jax.ShapeDtypeStruct((B_TOTAL, EMBED_DIM), jnp.float32),
    mesh=_mesh,
    scratch_types=[
        pltpu.VMEM((NCHUNK, CHUNK), jnp.int32),
        pltpu.VMEM((NBUF, CHUNK, EMBED_DIM), jnp.float32),
        pltpu.SemaphoreType.DMA,
    ],
    compiler_params=pltpu.CompilerParams(use_tc_tiling_on_sc=False),
)
def _lookup(idx_hbm, table_hbm, out_hbm, idx_v, rows_v, gsem):
    wid = lax.axis_index("s") * NUM_CORES + lax.axis_index("c")
    base = wid * ROWS_PER_W
    pltpu.sync_copy(idx_hbm.at[wid], idx_v)
    my_idx = idx_v

    pltpu.async_copy(table_hbm.at[my_idx.at[0]], rows_v.at[0], gsem)

    def outer(i, carry):
        for b in range(NBUF):
            j = i * NBUF + b
            nxt = j + 1

            @pl.when(nxt < NCHUNK)
            def _():
                pltpu.async_copy(
                    table_hbm.at[my_idx.at[nxt]],
                    rows_v.at[(b + 1) % NBUF],
                    gsem,
                )

            pltpu.make_async_copy(
                table_hbm.at[my_idx.at[j]], rows_v.at[b], gsem
            ).wait()
            pltpu.sync_copy(
                rows_v.at[b], out_hbm.at[pl.ds(base + j * CHUNK, CHUNK)]
            )
        return carry

    lax.fori_loop(0, NCHUNK // NBUF, outer, 0)


def kernel(token_indices, embeddings):
    idx = token_indices.reshape(NW, NCHUNK, CHUNK)
    out = _lookup(idx, embeddings)
    return out.reshape(BATCH, HIST, EMBED_DIM)


# speedup vs baseline: 3.4249x; 1.0088x over previous
import functools

import jax
import jax.numpy as jnp
from jax import lax
from jax.experimental import pallas as pl
from jax.experimental.pallas import tpu as pltpu
from jax.experimental.pallas import tpu_sc as plsc

VOCAB = 1000
EMBED_DIM = 64
BATCH = 1024
HIST = 200

NUM_CORES = 2
NUM_SUBCORES = 16
NW = NUM_CORES * NUM_SUBCORES
B_TOTAL = BATCH * HIST
ROWS_PER_W = B_TOTAL // NW
CHUNK = 320
NCHUNK = ROWS_PER_W // CHUNK
NBUF = 4

_mesh = plsc.VectorSubcoreMesh(core_axis_name="c", subcore_axis_name="s")


@functools.partial(
    pl.kernel,
    out_type=jax.ShapeDtypeStruct((B_TOTAL, EMBED_DIM), jnp.float32),
    mesh=_mesh,
    scratch_types=[
        pltpu.VMEM((NCHUNK, CHUNK), jnp.int32),
        pltpu.VMEM((NBUF, CHUNK, EMBED_DIM), jnp.float32),
        [pltpu.SemaphoreType.DMA] * NBUF,
    ],
    compiler_params=pltpu.CompilerParams(use_tc_tiling_on_sc=False),
)
def _lookup(idx_hbm, table_hbm, out_hbm, idx_v, rows_v, gsems):
    wid = lax.axis_index("s") * NUM_CORES + lax.axis_index("c")
    base = wid * ROWS_PER_W
    pltpu.sync_copy(idx_hbm.at[wid], idx_v)
    my_idx = idx_v

    for p in range(NBUF - 1):
        pltpu.async_copy(table_hbm.at[my_idx.at[p]], rows_v.at[p], gsems[p])

    def outer(i, carry):
        for b in range(NBUF):
            j = i * NBUF + b
            nxt = j + NBUF - 1

            @pl.when(nxt < NCHUNK)
            def _():
                pltpu.async_copy(
                    table_hbm.at[my_idx.at[nxt]],
                    rows_v.at[(b + NBUF - 1) % NBUF],
                    gsems[(b + NBUF - 1) % NBUF],
                )

            pltpu.make_async_copy(
                table_hbm.at[my_idx.at[j]], rows_v.at[b], gsems[b]
            ).wait()
            pltpu.sync_copy(
                rows_v.at[b], out_hbm.at[pl.ds(base + j * CHUNK, CHUNK)]
            )
        return carry

    lax.fori_loop(0, NCHUNK // NBUF, outer, 0)


def kernel(token_indices, embeddings):
    idx = token_indices.reshape(NW, NCHUNK, CHUNK)
    out = _lookup(idx, embeddings)
    return out.reshape(BATCH, HIST, EMBED_DIM)
